# Initial kernel scaffold; baseline (speedup 1.0000x reference)
import jax
import jax.numpy as jnp
from jax.experimental import pallas as pl


def kernel(train_x, test_x):
    raise NotImplementedError("write your pallas kernel here")


# speedup vs baseline: 1.9517x; 1.9517x over previous
import functools

import jax
import jax.numpy as jnp
from jax.experimental import pallas as pl
from jax.experimental.pallas import tpu as pltpu

K = 16
BIG = 3.0e38
IBIG = 2**31 - 1


def _knn_body(q_ref, t_ref, out_ref, vals_ref, idxs_ref, *, n_chunks, n_real, nb):
    j = pl.program_id(1)

    @pl.when(j == 0)
    def _init():
        vals_ref[...] = jnp.full(vals_ref.shape, BIG, jnp.float32)
        idxs_ref[...] = jnp.full(idxs_ref.shape, IBIG, jnp.int32)

    q = q_ref[...]
    t = t_ref[...]
    qb = q.shape[0]

    qs = jnp.sum(q * q, axis=1, keepdims=True)
    ts = jnp.sum(t * t, axis=1)
    cross = jax.lax.dot_general(
        q, t, (((1,), (1,)), ((), ())),
        preferred_element_type=jnp.float32)
    d = qs - 2.0 * cross + ts[None, :]

    base = j * nb
    col = base + jax.lax.broadcasted_iota(jnp.int32, (qb, nb), 1)
    d = jnp.where(col < n_real, d, BIG)

    mv = jnp.concatenate([vals_ref[...], d], axis=1)
    mi = jnp.concatenate([idxs_ref[...], col], axis=1)

    for r in range(K):
        m = jnp.min(mv, axis=1, keepdims=True)
        am = jnp.min(jnp.where(mv == m, mi, IBIG), axis=1, keepdims=True)
        vals_ref[:, r] = m[:, 0]
        idxs_ref[:, r] = am[:, 0]
        mv = jnp.where(mi == am, BIG, mv)

    @pl.when(j == n_chunks - 1)
    def _emit():
        out_ref[...] = idxs_ref[...]


@jax.jit
def kernel(train_x, test_x):
    n, dim = train_x.shape
    nq = test_x.shape[0]
    qb = 256
    nb = 2048
    n_pad = ((n + nb - 1) // nb) * nb
    n_chunks = n_pad // nb
    if n_pad != n:
        train_x = jnp.pad(train_x, ((0, n_pad - n), (0, 0)))

    grid = (nq // qb, n_chunks)
    out = pl.pallas_call(
        functools.partial(_knn_body, n_chunks=n_chunks, n_real=n, nb=nb),
        grid=grid,
        in_specs=[
            pl.BlockSpec((qb, dim), lambda i, j: (i, 0)),
            pl.BlockSpec((nb, dim), lambda i, j: (j, 0)),
        ],
        out_specs=pl.BlockSpec((qb, K), lambda i, j: (i, 0)),
        out_shape=jax.ShapeDtypeStruct((nq, K), jnp.int32),
        scratch_shapes=[
            pltpu.VMEM((qb, K), jnp.float32),
            pltpu.VMEM((qb, K), jnp.int32),
        ],
    )(test_x, train_x)
    return out.astype(jnp.int64)


# speedup vs baseline: 2.1163x; 1.0844x over previous
import functools

import jax
import jax.numpy as jnp
from jax import lax
from jax.experimental import pallas as pl
from jax.experimental.pallas import tpu as pltpu
from jax.experimental.pallas import tpu_sc as plsc

K = 16
G = 16
BIG = 3.0e38
IBIG = 2**31 - 1


def _p1_body(q_ref, tt_ref, m_ref, ts_ref, *, n_real, nb):
    j = pl.program_id(1)
    q = q_ref[...]
    tt = tt_ref[...]
    qb = q.shape[0]

    ts = jnp.sum(tt * tt, axis=0, keepdims=True)
    cross = jax.lax.dot_general(
        q, tt, (((1,), (0,)), ((), ())),
        preferred_element_type=jnp.float32)
    d = ts - 2.0 * cross
    col = j * nb + jax.lax.broadcasted_iota(jnp.int32, (qb, nb), 1)
    d = jnp.where(col < n_real, d, BIG)
    m_ref[...] = jnp.min(d.reshape(qb, nb // G, G), axis=2)
    ts_ref[...] = jnp.broadcast_to(ts, (8, ts.shape[1]))


def _p2_body(m_ref, g_ref, cv_ref, ci_ref, *, w, n_sub):
    c = pl.program_id(1)
    qb = m_ref.shape[0]

    @pl.when(c == 0)
    def _init():
        cv_ref[...] = jnp.full(cv_ref.shape, BIG, jnp.float32)
        ci_ref[...] = jnp.full(ci_ref.shape, IBIG, jnp.int32)

    mv = jnp.concatenate([cv_ref[...], m_ref[...]], axis=1)
    mi = jnp.concatenate(
        [ci_ref[...],
         c * w + jax.lax.broadcasted_iota(jnp.int32, (qb, w), 1)], axis=1)
    for r in range(K):
        m = jnp.min(mv, axis=1, keepdims=True)
        am = jnp.min(jnp.where(mv == m, mi, IBIG), axis=1, keepdims=True)
        cv_ref[:, r] = m[:, 0]
        ci_ref[:, r] = am[:, 0]
        mv = jnp.where(mi == am, BIG, mv)

    @pl.when(c == n_sub - 1)
    def _emit():
        g_ref[...] = ci_ref[...]


def _select_candidate_groups(train_x, test_x, qb, nb):
    n, dim = train_x.shape
    nq = test_x.shape[0]
    n_pad = ((n + nb - 1) // nb) * nb
    n_chunks = n_pad // nb
    n_groups = n_pad // G
    if n_pad != n:
        train_x = jnp.pad(train_x, ((0, n_pad - n), (0, 0)))
    train_t = train_x.T

    m, ts = pl.pallas_call(
        functools.partial(_p1_body, n_real=n, nb=nb),
        grid=(nq // qb, n_chunks),
        in_specs=[
            pl.BlockSpec((qb, dim), lambda i, j: (i, 0)),
            pl.BlockSpec((dim, nb), lambda i, j: (0, j)),
        ],
        out_specs=[
            pl.BlockSpec((qb, nb // G), lambda i, j: (i, j)),
            pl.BlockSpec((8, nb), lambda i, j: (0, j)),
        ],
        out_shape=[
            jax.ShapeDtypeStruct((nq, n_groups), jnp.float32),
            jax.ShapeDtypeStruct((8, n_pad), jnp.float32),
        ],
    )(test_x, train_t)

    n_sub = next(s for s in (7, 5, 4, 8, 2, 1)
                 if n_groups % s == 0 and (n_groups // s) % 128 == 0)
    w = n_groups // n_sub
    grp = pl.pallas_call(
        functools.partial(_p2_body, w=w, n_sub=n_sub),
        grid=(nq // qb, n_sub),
        in_specs=[pl.BlockSpec((qb, w), lambda i, c: (i, c))],
        out_specs=pl.BlockSpec((qb, K), lambda i, c: (i, 0)),
        out_shape=jax.ShapeDtypeStruct((nq, K), jnp.int32),
        scratch_shapes=[
            pltpu.VMEM((qb, K), jnp.float32),
            pltpu.VMEM((qb, K), jnp.int32),
        ],
    )(m)
    return grp, ts


def _round_bf16(x):
    u = jax.lax.bitcast_convert_type(x, jnp.uint32)
    r = (u + jnp.uint32(0x7FFF) + ((u >> 16) & jnp.uint32(1))) \
        & jnp.uint32(0xFFFF0000)
    return jax.lax.bitcast_convert_type(r, jnp.float32)


def _make_p3(nq, n, dim, n_groups):
    n_workers = 32
    qpt = nq // n_workers
    mesh = plsc.VectorSubcoreMesh(core_axis_name="c", subcore_axis_name="s")

    @functools.partial(
        pl.kernel, mesh=mesh,
        out_type=jax.ShapeDtypeStruct((nq * K,), jnp.int32),
        compiler_params=pltpu.CompilerParams(
            needs_layout_passes=False, use_tc_tiling_on_sc=False),
        scratch_types=[
            pltpu.VMEM((qpt * K,), jnp.int32),
            pltpu.VMEM((qpt * dim,), jnp.float32),
            pltpu.VMEM((K * G * dim,), jnp.float32),
            pltpu.VMEM((K * G,), jnp.float32),
            pltpu.VMEM((qpt * K,), jnp.int32),
            pltpu.SemaphoreType.DMA,
            pltpu.SemaphoreType.DMA,
        ])
    def p3(train_ref, grp_ref, test_ref, ts_ref, out_ref,
           grp_v, test_v, rows_v, ts_v, out_v, sem1, sem2):
        wid = lax.axis_index("s") * 2 + lax.axis_index("c")
        qbase = wid * qpt
        pltpu.sync_copy(grp_ref.at[pl.ds(qbase * K, qpt * K)], grp_v)
        pltpu.sync_copy(test_ref.at[pl.ds(qbase * dim, qpt * dim)], test_v)
        iota = lax.iota(jnp.int32, K)
        perms = [(iota + s) % K for s in (8, 4, 2, 1)]
        _dn = lax.GatherDimensionNumbers(
            offset_dims=(), collapsed_slice_dims=(0,), start_index_map=(0,))

        def _permute(x, idx):
            return lax.gather(
                x, idx[:, None], _dn, slice_sizes=(1,),
                mode=lax.GatherScatterMode.PROMISE_IN_BOUNDS)

        def _allsum(x):
            for p in perms:
                x = x + _permute(x, p)
            return x

        def do_query(q, carry):
            gvec = grp_v[pl.ds(q * K, K)]
            handles = []
            for j in range(K):
                gj = gvec[j]
                handles.append(pltpu.async_copy(
                    train_ref.at[gj], rows_v.at[pl.ds(j * G * dim, G * dim)],
                    sem1))
                handles.append(pltpu.async_copy(
                    ts_ref.at[gj], ts_v.at[pl.ds(j * G, G)], sem2))
            for h in handles:
                h.wait()

            q0 = test_v[pl.ds(q * dim, 16)]
            q1 = test_v[pl.ds(q * dim + 16, 16)]
            q2 = test_v[pl.ds(q * dim + 32, 16)]
            q3 = test_v[pl.ds(q * dim + 48, 16)]

            run_d = None
            for j in range(K):
                gj = gvec[j]
                tsvec = ts_v[pl.ds(j * G, G)]
                dj = jnp.full((K,), BIG, jnp.float32)
                for e in range(G):
                    c = (j * G + e) * dim
                    t0 = rows_v[pl.ds(c, 16)]
                    t1 = rows_v[pl.ds(c + 16, 16)]
                    t2 = rows_v[pl.ds(c + 32, 16)]
                    t3 = rows_v[pl.ds(c + 48, 16)]
                    prod = q0 * t0 + q1 * t1 + q2 * t2 + q3 * t3
                    cross = _allsum(prod)
                    tse = _permute(tsvec, jnp.full((K,), e, jnp.int32))
                    dj = jnp.where(iota == e, tse - 2.0 * cross, dj)
                cand = gj * G + iota
                dj = jnp.where(cand < n, dj, BIG)
                sd, si = plsc.sort_key_val(dj, cand)
                if run_d is None:
                    run_d, run_i = sd, si
                else:
                    rd = lax.rev(sd, (0,))
                    ri = lax.rev(si, (0,))
                    take = (run_d < rd) | ((run_d == rd) & (run_i < ri))
                    ld = jnp.where(take, run_d, rd)
                    li = jnp.where(take, run_i, ri)
                    run_d, run_i = plsc.sort_key_val(ld, li)
            out_v[pl.ds(q * K, K)] = run_i
            return carry

        lax.fori_loop(0, qpt, do_query, 0)
        pltpu.sync_copy(out_v, out_ref.at[pl.ds(qbase * K, qpt * K)])

    return p3


@jax.jit
def kernel(train_x, test_x):
    nq = test_x.shape[0]
    nb = 2048
    n, dim = train_x.shape
    grp, ts = _select_candidate_groups(train_x, test_x, min(256, nq), nb)

    n_pad = ts.shape[1]
    n_groups = n_pad // G
    train_p = jnp.pad(train_x, ((0, n_pad - n), (0, 0)))
    train_bf = _round_bf16(train_p)
    train_g = train_bf.reshape(n_groups, G * dim)
    test_bf = _round_bf16(test_x).reshape(-1)
    ts_g = ts[0].reshape(n_groups, G)
    grp_flat = grp.reshape(-1)

    out = _make_p3(nq, n, dim, n_groups)(train_g, grp_flat, test_bf, ts_g)
    return out.reshape(nq, K).astype(jnp.int64)
